# baseline (speedup 1.0000x reference)
import functools

import jax
import jax.numpy as jnp
from jax import lax
from jax.experimental import pallas as pl
from jax.experimental.pallas import tpu as pltpu
from jax.experimental.pallas import tpu_sc as plsc

N = 10000
E = 160000
D_IN = 256
HID = 1024
HID2 = 512
NCLS = 8

LANES = 16
NSC = 2
NTILE = 16
NPAD = 10240
CW = NPAD // NTILE
EPT = E // (NSC * NTILE)
EPS = E // NTILE
K = 125
KP = 128
SG = 2000
NG = EPS // SG
GB = SG // K
RPT = NPAD // NTILE

RB = 2048
GRID = 5


def _mesh():
    return plsc.VectorSubcoreMesh(core_axis_name="c", subcore_axis_name="s")


@functools.partial(
    pl.kernel,
    out_type=jax.ShapeDtypeStruct((NSC, NPAD), jnp.float32),
    mesh=_mesh(),
    compiler_params=pltpu.CompilerParams(needs_layout_passes=False),
    scratch_types=[
        pltpu.VMEM((EPT + LANES,), jnp.int32),
        pltpu.VMEM((NPAD,), jnp.float32),
        pltpu.VMEM((NTILE, CW), jnp.float32),
        pltpu.VMEM((CW,), jnp.float32),
        pltpu.VMEM_SHARED((NTILE, NPAD), jnp.float32),
    ],
)
def _deg(ei_hbm, out_hbm, dvm, hist, redbuf, sumbuf, shared):
    c = lax.axis_index("c")
    s = lax.axis_index("s")
    w = c * NTILE + s
    pltpu.sync_copy(ei_hbm.at[pl.ds(E + w * EPT, EPT)], dvm.at[pl.ds(0, EPT)])

    zv = jnp.zeros((LANES,), jnp.float32)

    def zbody(i, _):
        hist[pl.ds(i * LANES, LANES)] = zv
        return 0

    lax.fori_loop(0, NPAD // LANES, zbody, 0)

    ones = jnp.ones((LANES,), jnp.float32)
    nfull = EPT // LANES

    def abody(i, _):
        idx = dvm[pl.ds(i * LANES, LANES)]
        plsc.addupdate_scatter(hist, [idx], ones)
        return 0

    lax.fori_loop(0, nfull, abody, 0)
    rem = EPT - nfull * LANES
    if rem:
        lanemask = lax.iota(jnp.int32, LANES) < rem
        idx = dvm[pl.ds(nfull * LANES, LANES)]
        idx = jnp.where(lanemask, idx, 0)
        plsc.addupdate_scatter(hist, [idx], ones, mask=lanemask)

    pltpu.sync_copy(hist, shared.at[s])
    plsc.subcore_barrier()
    for r in range(NTILE):
        pltpu.sync_copy(shared.at[r, pl.ds(s * CW, CW)], redbuf.at[r])

    def rbody(k, _):
        acc = redbuf[0, pl.ds(k * LANES, LANES)]
        for r in range(1, NTILE):
            acc = acc + redbuf[r, pl.ds(k * LANES, LANES)]
        sumbuf[pl.ds(k * LANES, LANES)] = acc
        return 0

    lax.fori_loop(0, CW // LANES, rbody, 0)
    pltpu.sync_copy(sumbuf, out_hbm.at[c, pl.ds(s * CW, CW)])


def _make_agg(num_chunks):
    cpc = num_chunks // NSC

    @functools.partial(
        pl.kernel,
        out_type=jax.ShapeDtypeStruct((num_chunks * NPAD, 128), jnp.float32),
        mesh=_mesh(),
        scratch_types=[
            pltpu.VMEM((SG + LANES,), jnp.int32),
            pltpu.VMEM((SG + LANES,), jnp.int32),
            pltpu.VMEM((KP,), jnp.int32),
            pltpu.VMEM((KP,), jnp.int32),
            pltpu.VMEM((KP,), jnp.int32),
            pltpu.VMEM((KP,), jnp.int32),
            pltpu.VMEM((KP, 128), jnp.float32),
            pltpu.VMEM((KP, 128), jnp.float32),
            pltpu.VMEM_SHARED((NPAD, 128), jnp.float32),
            pltpu.SemaphoreType.DMA,
            pltpu.SemaphoreType.DMA,
        ],
    )
    def agg(y_hbm, ei_hbm, z_hbm, out_hbm, sflat, dflat, si0, di0, si1, di1,
            gb0, gb1, acc, sm0, sm1):
        c = lax.axis_index("c")
        s = lax.axis_index("s")
        sx = [si0, si1]
        dx = [di0, di1]
        gb = [gb0, gb1]
        sm = [sm0, sm1]
        lane = lax.iota(jnp.int32, LANES)
        dumdst = jnp.int32(N) + s

        for p in range(cpc):
            chunk = c * cpc + p
            base = chunk * NPAD

            def fill(j, t):
                e0 = j * K
                nv = K - (KP // LANES - 1) * LANES
                for k in range(KP // LANES):
                    sv = sflat[pl.ds(e0 + k * LANES, LANES)]
                    dv = dflat[pl.ds(e0 + k * LANES, LANES)]
                    if k == KP // LANES - 1:
                        valid = lane < nv
                        sv = jnp.where(valid, sv, 0)
                        dv = jnp.where(valid, dv, dumdst)
                    sx[t][pl.ds(k * LANES, LANES)] = sv + base
                    dx[t][pl.ds(k * LANES, LANES)] = dv

            def issue(t):
                pltpu.async_copy(y_hbm.at[sx[t]], gb[t], sm[t])

            def drain(t):
                pltpu.make_async_copy(y_hbm.at[sx[t]], gb[t], sm[t]).wait()
                pltpu.sync_copy(gb[t], acc.at[dx[t]], add=True)

            pltpu.sync_copy(z_hbm.at[pl.ds(s * RPT, RPT)],
                            acc.at[pl.ds(s * RPT, RPT)])
            plsc.subcore_barrier()

            for g in range(NG):
                g0 = s * EPS + g * SG
                pltpu.sync_copy(ei_hbm.at[pl.ds(g0, SG)],
                                sflat.at[pl.ds(0, SG)])
                pltpu.sync_copy(ei_hbm.at[pl.ds(E + g0, SG)],
                                dflat.at[pl.ds(0, SG)])
                for t in range(2):
                    fill(t, t)
                    issue(t)

                def ibody(m, _):
                    for t in range(2):
                        drain(t)
                        fill(2 * m + t + 2, t)
                        issue(t)
                    return 0

                lax.fori_loop(0, GB // 2 - 1, ibody, 0)
                drain(0)
                drain(1)

            plsc.subcore_barrier()
            pltpu.sync_copy(
                acc.at[pl.ds(s * RPT, RPT)],
                out_hbm.at[pl.ds(base + s * RPT, RPT)])

    return agg


_agg2 = _make_agg(2)
_agg4 = _make_agg(4)


def _prep_body(x_ref, dp_ref, y1_ref, dis_ref):
    d = dp_ref[0, :] + dp_ref[1, :] + 1.0
    dis = 1.0 / jnp.sqrt(d)
    dis_ref[...] = dis
    xb = x_ref[...]
    y1_ref[0] = xb[:, :128] * dis[:, None]
    y1_ref[1] = xb[:, 128:] * dis[:, None]


_prep = pl.pallas_call(
    _prep_body,
    grid=(GRID,),
    in_specs=[
        pl.BlockSpec((RB, D_IN), lambda i: (i, 0)),
        pl.BlockSpec((NSC, RB), lambda i: (0, i)),
    ],
    out_specs=[
        pl.BlockSpec((NSC, RB, 128), lambda i: (0, i, 0)),
        pl.BlockSpec((RB,), lambda i: (i,)),
    ],
    out_shape=[
        jax.ShapeDtypeStruct((NSC, NPAD, 128), jnp.float32),
        jax.ShapeDtypeStruct((N,), jnp.float32),
    ],
)


def _mid_body(a1_ref, y1_ref, dis_ref, w1_ref, b1_ref, w2_ref, y2_ref):
    dis = dis_ref[...][:, None]
    z = jnp.concatenate(
        [(a1_ref[0] + y1_ref[0]) * dis, (a1_ref[1] + y1_ref[1]) * dis],
        axis=1).astype(jnp.bfloat16)
    h = jnp.dot(z, w1_ref[...], preferred_element_type=jnp.float32) + b1_ref[...][None, :]
    h = jnp.maximum(h, 0.0).astype(jnp.bfloat16)
    t = jnp.dot(h, w2_ref[...], preferred_element_type=jnp.float32)
    for q in range(4):
        y2_ref[q] = t[:, q * 128:(q + 1) * 128] * dis


_mid = pl.pallas_call(
    _mid_body,
    grid=(GRID,),
    in_specs=[
        pl.BlockSpec((NSC, RB, 128), lambda i: (0, i, 0)),
        pl.BlockSpec((NSC, RB, 128), lambda i: (0, i, 0)),
        pl.BlockSpec((RB,), lambda i: (i,)),
        pl.BlockSpec((D_IN, HID), lambda i: (0, 0)),
        pl.BlockSpec((HID,), lambda i: (0,)),
        pl.BlockSpec((HID, HID2), lambda i: (0, 0)),
    ],
    out_specs=pl.BlockSpec((4, RB, 128), lambda i: (0, i, 0)),
    out_shape=jax.ShapeDtypeStruct((4, NPAD, 128), jnp.float32),
)


def _fin_body(a2_ref, y2_ref, dis_ref, b2_ref, wc_ref, bc_ref, out_ref):
    dis = dis_ref[...][:, None]
    z = jnp.concatenate(
        [(a2_ref[q] + y2_ref[q]) * dis for q in range(4)], axis=1)
    h = jnp.maximum(z + b2_ref[...][None, :], 0.0)
    logits = jnp.dot(h, wc_ref[...], preferred_element_type=jnp.float32) + bc_ref[...][None, :]
    m = jnp.max(logits, axis=1, keepdims=True)
    ex = jnp.exp(logits - m)
    lse = jnp.log(jnp.sum(ex, axis=1, keepdims=True)) + m
    out_ref[...] = logits - lse


_fin = pl.pallas_call(
    _fin_body,
    grid=(GRID,),
    in_specs=[
        pl.BlockSpec((4, RB, 128), lambda i: (0, i, 0)),
        pl.BlockSpec((4, RB, 128), lambda i: (0, i, 0)),
        pl.BlockSpec((RB,), lambda i: (i,)),
        pl.BlockSpec((HID2,), lambda i: (0,)),
        pl.BlockSpec((HID2, NCLS), lambda i: (0, 0)),
        pl.BlockSpec((NCLS,), lambda i: (0,)),
    ],
    out_specs=pl.BlockSpec((RB, NCLS), lambda i: (i, 0)),
    out_shape=jax.ShapeDtypeStruct((N, NCLS), jnp.float32),
)


def kernel(x, edge_index, W1, b1, W2, b2, Wc, bc):
    ei_flat = edge_index.reshape(2 * E)
    zeros = jnp.zeros((NPAD, 128), jnp.float32)
    deg_p = _deg(ei_flat)
    y1, dis = _prep(x, deg_p)
    agg1 = _agg2(y1.reshape(NSC * NPAD, 128), ei_flat, zeros)
    y2 = _mid(agg1.reshape(NSC, NPAD, 128), y1, dis,
              W1.astype(jnp.bfloat16), b1, W2.astype(jnp.bfloat16))
    agg2 = _agg4(y2.reshape(4 * NPAD, 128), ei_flat, zeros)
    return _fin(agg2.reshape(4, NPAD, 128), y2, dis, b2, Wc, bc)


# speedup vs baseline: 1.9238x; 1.9238x over previous
import functools

import jax
import jax.numpy as jnp
from jax import lax
from jax.experimental import pallas as pl
from jax.experimental.pallas import tpu as pltpu
from jax.experimental.pallas import tpu_sc as plsc

N = 10000
E = 160000
D_IN = 256
HID = 1024
HID2 = 512
NCLS = 8

LANES = 16
NSC = 2
NTILE = 16
NPAD = 10240
CW = NPAD // NTILE
EPT = E // (NSC * NTILE)
EPS = E // NTILE
K = 80
SG = 2000
NG = EPS // SG
GB = SG // K
RPT = NPAD // NTILE

RB = 2048
GRID = 5


def _mesh():
    return plsc.VectorSubcoreMesh(core_axis_name="c", subcore_axis_name="s")


@functools.partial(
    pl.kernel,
    out_type=jax.ShapeDtypeStruct((NSC, NPAD), jnp.float32),
    mesh=_mesh(),
    compiler_params=pltpu.CompilerParams(needs_layout_passes=False),
    scratch_types=[
        pltpu.VMEM((EPT + LANES,), jnp.int32),
        pltpu.VMEM((NPAD,), jnp.float32),
        pltpu.VMEM((NTILE, CW), jnp.float32),
        pltpu.VMEM((CW,), jnp.float32),
        pltpu.VMEM_SHARED((NTILE, NPAD), jnp.float32),
    ],
)
def _deg(ei_hbm, out_hbm, dvm, hist, redbuf, sumbuf, shared):
    c = lax.axis_index("c")
    s = lax.axis_index("s")
    w = c * NTILE + s
    pltpu.sync_copy(ei_hbm.at[pl.ds(E + w * EPT, EPT)], dvm.at[pl.ds(0, EPT)])

    zv = jnp.zeros((LANES,), jnp.float32)

    def zbody(i, _):
        hist[pl.ds(i * LANES, LANES)] = zv
        return 0

    lax.fori_loop(0, NPAD // LANES, zbody, 0)

    ones = jnp.ones((LANES,), jnp.float32)
    nfull = EPT // LANES

    def abody(i, _):
        idx = dvm[pl.ds(i * LANES, LANES)]
        plsc.addupdate_scatter(hist, [idx], ones)
        return 0

    lax.fori_loop(0, nfull, abody, 0)
    rem = EPT - nfull * LANES
    if rem:
        lanemask = lax.iota(jnp.int32, LANES) < rem
        idx = dvm[pl.ds(nfull * LANES, LANES)]
        idx = jnp.where(lanemask, idx, 0)
        plsc.addupdate_scatter(hist, [idx], ones, mask=lanemask)

    pltpu.sync_copy(hist, shared.at[s])
    plsc.subcore_barrier()
    for r in range(NTILE):
        pltpu.sync_copy(shared.at[r, pl.ds(s * CW, CW)], redbuf.at[r])

    def rbody(k, _):
        acc = redbuf[0, pl.ds(k * LANES, LANES)]
        for r in range(1, NTILE):
            acc = acc + redbuf[r, pl.ds(k * LANES, LANES)]
        sumbuf[pl.ds(k * LANES, LANES)] = acc
        return 0

    lax.fori_loop(0, CW // LANES, rbody, 0)
    pltpu.sync_copy(sumbuf, out_hbm.at[c, pl.ds(s * CW, CW)])


def _make_agg(num_chunks):
    cpc = num_chunks // NSC

    @functools.partial(
        pl.kernel,
        out_type=jax.ShapeDtypeStruct((num_chunks * NPAD, 128), jnp.float32),
        mesh=_mesh(),
        scratch_types=[
            pltpu.VMEM((SG,), jnp.int32),
            pltpu.VMEM((SG,), jnp.int32),
            pltpu.VMEM((K,), jnp.int32),
            pltpu.VMEM((K,), jnp.int32),
            pltpu.VMEM((K,), jnp.int32),
            pltpu.VMEM((K,), jnp.int32),
            pltpu.VMEM((K,), jnp.int32),
            pltpu.VMEM((K,), jnp.int32),
            pltpu.VMEM((K,), jnp.int32),
            pltpu.VMEM((K,), jnp.int32),
            pltpu.VMEM((K, 128), jnp.float32),
            pltpu.VMEM((K, 128), jnp.float32),
            pltpu.VMEM((K, 128), jnp.float32),
            pltpu.VMEM((K, 128), jnp.float32),
            pltpu.VMEM_SHARED((NPAD, 128), jnp.float32),
            pltpu.SemaphoreType.DMA,
            pltpu.SemaphoreType.DMA,
            pltpu.SemaphoreType.DMA,
            pltpu.SemaphoreType.DMA,
        ],
    )
    def agg(y_hbm, ei_hbm, z_hbm, out_hbm, sflat, dflat, si0, di0, si1, di1,
            si2, di2, si3, di3, gb0, gb1, gb2, gb3, acc, sm0, sm1, sm2, sm3):
        c = lax.axis_index("c")
        s = lax.axis_index("s")
        sx = [si0, si1, si2, si3]
        dx = [di0, di1, di2, di3]
        gb = [gb0, gb1, gb2, gb3]
        sm = [sm0, sm1, sm2, sm3]

        for p in range(cpc):
            chunk = c * cpc + p
            base = chunk * NPAD

            def fill(j, t):
                e0 = j * K
                for k in range(K // LANES):
                    sv = sflat[pl.ds(e0 + k * LANES, LANES)]
                    sx[t][pl.ds(k * LANES, LANES)] = sv + base
                    dx[t][pl.ds(k * LANES, LANES)] = (
                        dflat[pl.ds(e0 + k * LANES, LANES)])

            def issue(t):
                pltpu.async_copy(y_hbm.at[sx[t]], gb[t], sm[t])

            def drain(t):
                pltpu.make_async_copy(y_hbm.at[sx[t]], gb[t], sm[t]).wait()
                pltpu.sync_copy(gb[t], acc.at[dx[t]], add=True)

            pltpu.sync_copy(z_hbm.at[pl.ds(s * RPT, RPT)],
                            acc.at[pl.ds(s * RPT, RPT)])
            plsc.subcore_barrier()

            for g in range(NG):
                g0 = s * EPS + g * SG
                pltpu.sync_copy(ei_hbm.at[pl.ds(g0, SG)], sflat)
                pltpu.sync_copy(ei_hbm.at[pl.ds(E + g0, SG)], dflat)
                for t in range(4):
                    fill(t, t)
                    issue(t)

                def ibody(m, _):
                    for t in range(4):
                        drain(t)
                        fill(4 * m + t + 4, t)
                        issue(t)
                    return 0

                lax.fori_loop(0, (GB - 5) // 4, ibody, 0)
                drain(0)
                fill(GB - 1, 0)
                issue(0)
                drain(1)
                drain(2)
                drain(3)
                drain(0)

            plsc.subcore_barrier()
            pltpu.sync_copy(
                acc.at[pl.ds(s * RPT, RPT)],
                out_hbm.at[pl.ds(base + s * RPT, RPT)])

    return agg


_agg2 = _make_agg(2)
_agg4 = _make_agg(4)


def _prep_body(x_ref, dp_ref, y1_ref, dis_ref):
    d = dp_ref[0, :] + dp_ref[1, :] + 1.0
    dis = 1.0 / jnp.sqrt(d)
    dis_ref[...] = dis
    xb = x_ref[...]
    y1_ref[0] = xb[:, :128] * dis[:, None]
    y1_ref[1] = xb[:, 128:] * dis[:, None]


_prep = pl.pallas_call(
    _prep_body,
    grid=(GRID,),
    in_specs=[
        pl.BlockSpec((RB, D_IN), lambda i: (i, 0)),
        pl.BlockSpec((NSC, RB), lambda i: (0, i)),
    ],
    out_specs=[
        pl.BlockSpec((NSC, RB, 128), lambda i: (0, i, 0)),
        pl.BlockSpec((RB,), lambda i: (i,)),
    ],
    out_shape=[
        jax.ShapeDtypeStruct((NSC, NPAD, 128), jnp.float32),
        jax.ShapeDtypeStruct((N,), jnp.float32),
    ],
)


def _mid_body(a1_ref, y1_ref, dis_ref, w1_ref, b1_ref, w2_ref, y2_ref):
    dis = dis_ref[...][:, None]
    z = jnp.concatenate(
        [(a1_ref[0] + y1_ref[0]) * dis, (a1_ref[1] + y1_ref[1]) * dis],
        axis=1).astype(jnp.bfloat16)
    h = jnp.dot(z, w1_ref[...], preferred_element_type=jnp.float32) + b1_ref[...][None, :]
    h = jnp.maximum(h, 0.0).astype(jnp.bfloat16)
    t = jnp.dot(h, w2_ref[...], preferred_element_type=jnp.float32)
    for q in range(4):
        y2_ref[q] = t[:, q * 128:(q + 1) * 128] * dis


_mid = pl.pallas_call(
    _mid_body,
    grid=(GRID,),
    in_specs=[
        pl.BlockSpec((NSC, RB, 128), lambda i: (0, i, 0)),
        pl.BlockSpec((NSC, RB, 128), lambda i: (0, i, 0)),
        pl.BlockSpec((RB,), lambda i: (i,)),
        pl.BlockSpec((D_IN, HID), lambda i: (0, 0)),
        pl.BlockSpec((HID,), lambda i: (0,)),
        pl.BlockSpec((HID, HID2), lambda i: (0, 0)),
    ],
    out_specs=pl.BlockSpec((4, RB, 128), lambda i: (0, i, 0)),
    out_shape=jax.ShapeDtypeStruct((4, NPAD, 128), jnp.float32),
)


def _fin_body(a2_ref, y2_ref, dis_ref, b2_ref, wc_ref, bc_ref, out_ref):
    dis = dis_ref[...][:, None]
    z = jnp.concatenate(
        [(a2_ref[q] + y2_ref[q]) * dis for q in range(4)], axis=1)
    h = jnp.maximum(z + b2_ref[...][None, :], 0.0)
    logits = jnp.dot(h, wc_ref[...], preferred_element_type=jnp.float32) + bc_ref[...][None, :]
    m = jnp.max(logits, axis=1, keepdims=True)
    ex = jnp.exp(logits - m)
    lse = jnp.log(jnp.sum(ex, axis=1, keepdims=True)) + m
    out_ref[...] = logits - lse


_fin = pl.pallas_call(
    _fin_body,
    grid=(GRID,),
    in_specs=[
        pl.BlockSpec((4, RB, 128), lambda i: (0, i, 0)),
        pl.BlockSpec((4, RB, 128), lambda i: (0, i, 0)),
        pl.BlockSpec((RB,), lambda i: (i,)),
        pl.BlockSpec((HID2,), lambda i: (0,)),
        pl.BlockSpec((HID2, NCLS), lambda i: (0, 0)),
        pl.BlockSpec((NCLS,), lambda i: (0,)),
    ],
    out_specs=pl.BlockSpec((RB, NCLS), lambda i: (i, 0)),
    out_shape=jax.ShapeDtypeStruct((N, NCLS), jnp.float32),
)


def kernel(x, edge_index, W1, b1, W2, b2, Wc, bc):
    ei_flat = edge_index.reshape(2 * E)
    zeros = jnp.zeros((NPAD, 128), jnp.float32)
    deg_p = _deg(ei_flat)
    y1, dis = _prep(x, deg_p)
    agg1 = _agg2(y1.reshape(NSC * NPAD, 128), ei_flat, zeros)
    y2 = _mid(agg1.reshape(NSC, NPAD, 128), y1, dis,
              W1.astype(jnp.bfloat16), b1, W2.astype(jnp.bfloat16))
    agg2 = _agg4(y2.reshape(4 * NPAD, 128), ei_flat, zeros)
    return _fin(agg2.reshape(4, NPAD, 128), y2, dis, b2, Wc, bc)


# speedup vs baseline: 1.9634x; 1.0206x over previous
import functools

import jax
import jax.numpy as jnp
from jax import lax
from jax.experimental import pallas as pl
from jax.experimental.pallas import tpu as pltpu
from jax.experimental.pallas import tpu_sc as plsc

N = 10000
E = 160000
D_IN = 256
HID = 1024
HID2 = 512
NCLS = 8

LANES = 16
NSC = 2
NTILE = 16
NPAD = 10240
CW = NPAD // NTILE
EPT = E // (NSC * NTILE)
EPS = E // NTILE
K = 80
SG = 2000
NG = EPS // SG
GB = SG // K
RPT = NPAD // NTILE

RB = 2048
GRID = 5


def _mesh():
    return plsc.VectorSubcoreMesh(core_axis_name="c", subcore_axis_name="s")


@functools.partial(
    pl.kernel,
    out_type=jax.ShapeDtypeStruct((NSC, NPAD), jnp.float32),
    mesh=_mesh(),
    compiler_params=pltpu.CompilerParams(needs_layout_passes=False),
    scratch_types=[
        pltpu.VMEM((EPT + LANES,), jnp.int32),
        pltpu.VMEM((NPAD,), jnp.float32),
        pltpu.VMEM((NTILE, CW), jnp.float32),
        pltpu.VMEM((CW,), jnp.float32),
        pltpu.VMEM_SHARED((NTILE, NPAD), jnp.float32),
    ],
)
def _deg(ei_hbm, out_hbm, dvm, hist, redbuf, sumbuf, shared):
    c = lax.axis_index("c")
    s = lax.axis_index("s")
    w = c * NTILE + s
    pltpu.sync_copy(ei_hbm.at[pl.ds(E + w * EPT, EPT)], dvm.at[pl.ds(0, EPT)])

    zv = jnp.zeros((LANES,), jnp.float32)

    def zbody(i, _):
        hist[pl.ds(i * LANES, LANES)] = zv
        return 0

    lax.fori_loop(0, NPAD // LANES, zbody, 0)

    ones = jnp.ones((LANES,), jnp.float32)
    nfull = EPT // LANES

    def abody(i, _):
        idx = dvm[pl.ds(i * LANES, LANES)]
        plsc.addupdate_scatter(hist, [idx], ones)
        return 0

    lax.fori_loop(0, nfull, abody, 0)
    rem = EPT - nfull * LANES
    if rem:
        lanemask = lax.iota(jnp.int32, LANES) < rem
        idx = dvm[pl.ds(nfull * LANES, LANES)]
        idx = jnp.where(lanemask, idx, 0)
        plsc.addupdate_scatter(hist, [idx], ones, mask=lanemask)

    pltpu.sync_copy(hist, shared.at[s])
    plsc.subcore_barrier()
    for r in range(NTILE):
        pltpu.sync_copy(shared.at[r, pl.ds(s * CW, CW)], redbuf.at[r])

    def rbody(k, _):
        acc = redbuf[0, pl.ds(k * LANES, LANES)]
        for r in range(1, NTILE):
            acc = acc + redbuf[r, pl.ds(k * LANES, LANES)]
        sumbuf[pl.ds(k * LANES, LANES)] = acc
        return 0

    lax.fori_loop(0, CW // LANES, rbody, 0)
    pltpu.sync_copy(sumbuf, out_hbm.at[c, pl.ds(s * CW, CW)])


def _make_agg(num_chunks):
    cpc = num_chunks // NSC

    @functools.partial(
        pl.kernel,
        out_type=jax.ShapeDtypeStruct((num_chunks * NPAD, 128), jnp.float32),
        mesh=_mesh(),
        scratch_types=[
            pltpu.VMEM((SG,), jnp.int32),
            pltpu.VMEM((SG,), jnp.int32),
            pltpu.VMEM((K,), jnp.int32),
            pltpu.VMEM((K,), jnp.int32),
            pltpu.VMEM((K,), jnp.int32),
            pltpu.VMEM((K,), jnp.int32),
            pltpu.VMEM((K,), jnp.int32),
            pltpu.VMEM((K,), jnp.int32),
            pltpu.VMEM((K,), jnp.int32),
            pltpu.VMEM((K,), jnp.int32),
            pltpu.VMEM((K, 128), jnp.float32),
            pltpu.VMEM((K, 128), jnp.float32),
            pltpu.VMEM((K, 128), jnp.float32),
            pltpu.VMEM((K, 128), jnp.float32),
            pltpu.VMEM_SHARED((NPAD, 128), jnp.float32),
            pltpu.SemaphoreType.DMA,
            pltpu.SemaphoreType.DMA,
            pltpu.SemaphoreType.DMA,
            pltpu.SemaphoreType.DMA,
            pltpu.SemaphoreType.DMA,
        ],
    )
    def agg(y_hbm, ei_hbm, z_hbm, out_hbm, sflat, dflat, si0, di0, si1, di1,
            si2, di2, si3, di3, gb0, gb1, gb2, gb3, acc, sm0, sm1, sm2, sm3,
            zsem):
        c = lax.axis_index("c")
        s = lax.axis_index("s")
        sx = [si0, si1, si2, si3]
        dx = [di0, di1, di2, di3]
        gb = [gb0, gb1, gb2, gb3]
        sm = [sm0, sm1, sm2, sm3]

        for p in range(cpc):
            chunk = c * cpc + p
            base = chunk * NPAD

            def fill(j, t):
                e0 = j * K
                for k in range(K // LANES):
                    sv = sflat[pl.ds(e0 + k * LANES, LANES)]
                    sx[t][pl.ds(k * LANES, LANES)] = sv + base
                    dx[t][pl.ds(k * LANES, LANES)] = (
                        dflat[pl.ds(e0 + k * LANES, LANES)])

            def issue(t):
                pltpu.async_copy(y_hbm.at[sx[t]], gb[t], sm[t])

            def drain(t):
                pltpu.make_async_copy(y_hbm.at[sx[t]], gb[t], sm[t]).wait()
                pltpu.sync_copy(gb[t], acc.at[dx[t]], add=True)

            zdesc = pltpu.async_copy(z_hbm.at[pl.ds(s * RPT, RPT)],
                                     acc.at[pl.ds(s * RPT, RPT)], zsem)

            for g in range(NG):
                g0 = s * EPS + g * SG
                pltpu.sync_copy(ei_hbm.at[pl.ds(g0, SG)], sflat)
                pltpu.sync_copy(ei_hbm.at[pl.ds(E + g0, SG)], dflat)
                for t in range(4):
                    fill(t, t)
                    issue(t)
                if g == 0:
                    zdesc.wait()
                    plsc.subcore_barrier()

                def ibody(m, _):
                    for t in range(4):
                        drain(t)
                        fill(4 * m + t + 4, t)
                        issue(t)
                    return 0

                lax.fori_loop(0, (GB - 5) // 4, ibody, 0)
                drain(0)
                fill(GB - 1, 0)
                issue(0)
                drain(1)
                drain(2)
                drain(3)
                drain(0)

            plsc.subcore_barrier()
            pltpu.sync_copy(
                acc.at[pl.ds(s * RPT, RPT)],
                out_hbm.at[pl.ds(base + s * RPT, RPT)])

    return agg


_agg2 = _make_agg(2)
_agg4 = _make_agg(4)


def _prep_body(x_ref, dp_ref, y1_ref, dis_ref):
    d = dp_ref[0, :] + dp_ref[1, :] + 1.0
    dis = 1.0 / jnp.sqrt(d)
    dis_ref[...] = dis
    xb = x_ref[...]
    y1_ref[0] = xb[:, :128] * dis[:, None]
    y1_ref[1] = xb[:, 128:] * dis[:, None]


_prep = pl.pallas_call(
    _prep_body,
    grid=(GRID,),
    in_specs=[
        pl.BlockSpec((RB, D_IN), lambda i: (i, 0)),
        pl.BlockSpec((NSC, RB), lambda i: (0, i)),
    ],
    out_specs=[
        pl.BlockSpec((NSC, RB, 128), lambda i: (0, i, 0)),
        pl.BlockSpec((RB,), lambda i: (i,)),
    ],
    out_shape=[
        jax.ShapeDtypeStruct((NSC, NPAD, 128), jnp.float32),
        jax.ShapeDtypeStruct((N,), jnp.float32),
    ],
)


def _mid_body(a1_ref, y1_ref, dis_ref, w1_ref, b1_ref, w2_ref, y2_ref):
    dis = dis_ref[...][:, None]
    z = jnp.concatenate(
        [(a1_ref[0] + y1_ref[0]) * dis, (a1_ref[1] + y1_ref[1]) * dis],
        axis=1).astype(jnp.bfloat16)
    h = jnp.dot(z, w1_ref[...], preferred_element_type=jnp.float32) + b1_ref[...][None, :]
    h = jnp.maximum(h, 0.0).astype(jnp.bfloat16)
    t = jnp.dot(h, w2_ref[...], preferred_element_type=jnp.float32)
    for q in range(4):
        y2_ref[q] = t[:, q * 128:(q + 1) * 128] * dis


_mid = pl.pallas_call(
    _mid_body,
    grid=(GRID,),
    in_specs=[
        pl.BlockSpec((NSC, RB, 128), lambda i: (0, i, 0)),
        pl.BlockSpec((NSC, RB, 128), lambda i: (0, i, 0)),
        pl.BlockSpec((RB,), lambda i: (i,)),
        pl.BlockSpec((D_IN, HID), lambda i: (0, 0)),
        pl.BlockSpec((HID,), lambda i: (0,)),
        pl.BlockSpec((HID, HID2), lambda i: (0, 0)),
    ],
    out_specs=pl.BlockSpec((4, RB, 128), lambda i: (0, i, 0)),
    out_shape=jax.ShapeDtypeStruct((4, NPAD, 128), jnp.float32),
)


def _fin_body(a2_ref, y2_ref, dis_ref, b2_ref, wc_ref, bc_ref, out_ref):
    dis = dis_ref[...][:, None]
    z = jnp.concatenate(
        [(a2_ref[q] + y2_ref[q]) * dis for q in range(4)], axis=1)
    h = jnp.maximum(z + b2_ref[...][None, :], 0.0)
    logits = jnp.dot(h, wc_ref[...], preferred_element_type=jnp.float32) + bc_ref[...][None, :]
    m = jnp.max(logits, axis=1, keepdims=True)
    ex = jnp.exp(logits - m)
    lse = jnp.log(jnp.sum(ex, axis=1, keepdims=True)) + m
    out_ref[...] = logits - lse


_fin = pl.pallas_call(
    _fin_body,
    grid=(GRID,),
    in_specs=[
        pl.BlockSpec((4, RB, 128), lambda i: (0, i, 0)),
        pl.BlockSpec((4, RB, 128), lambda i: (0, i, 0)),
        pl.BlockSpec((RB,), lambda i: (i,)),
        pl.BlockSpec((HID2,), lambda i: (0,)),
        pl.BlockSpec((HID2, NCLS), lambda i: (0, 0)),
        pl.BlockSpec((NCLS,), lambda i: (0,)),
    ],
    out_specs=pl.BlockSpec((RB, NCLS), lambda i: (i, 0)),
    out_shape=jax.ShapeDtypeStruct((N, NCLS), jnp.float32),
)


def kernel(x, edge_index, W1, b1, W2, b2, Wc, bc):
    ei_flat = edge_index.reshape(2 * E)
    zeros = jnp.zeros((NPAD, 128), jnp.float32)
    deg_p = _deg(ei_flat)
    y1, dis = _prep(x, deg_p)
    agg1 = _agg2(y1.reshape(NSC * NPAD, 128), ei_flat, zeros)
    y2 = _mid(agg1.reshape(NSC, NPAD, 128), y1, dis,
              W1.astype(jnp.bfloat16), b1, W2.astype(jnp.bfloat16))
    agg2 = _agg4(y2.reshape(4 * NPAD, 128), ei_flat, zeros)
    return _fin(agg2.reshape(4, NPAD, 128), y2, dis, b2, Wc, bc)


# speedup vs baseline: 2.0420x; 1.0400x over previous
import functools

import jax
import jax.numpy as jnp
from jax import lax
from jax.experimental import pallas as pl
from jax.experimental.pallas import tpu as pltpu
from jax.experimental.pallas import tpu_sc as plsc

N = 10000
E = 160000
D_IN = 256
HID = 1024
HID2 = 512
NCLS = 8

LANES = 16
NSC = 2
NTILE = 16
NPAD = 10240
CW = NPAD // NTILE
EPT = E // (NSC * NTILE)
EPS = E // NTILE
K = 80
SG = 2000
NG = EPS // SG
GB = SG // K
RPT = NPAD // NTILE

RB = 2048
GRID = 5


def _mesh():
    return plsc.VectorSubcoreMesh(core_axis_name="c", subcore_axis_name="s")


@functools.partial(
    pl.kernel,
    out_type=jax.ShapeDtypeStruct((NSC, NPAD), jnp.float32),
    mesh=_mesh(),
    compiler_params=pltpu.CompilerParams(needs_layout_passes=False),
    scratch_types=[
        pltpu.VMEM((EPT + LANES,), jnp.int32),
        pltpu.VMEM((NPAD,), jnp.float32),
        pltpu.VMEM((NTILE, CW), jnp.float32),
        pltpu.VMEM((CW,), jnp.float32),
        pltpu.VMEM_SHARED((NTILE, NPAD), jnp.float32),
    ],
)
def _deg(ei_hbm, out_hbm, dvm, hist, redbuf, sumbuf, shared):
    c = lax.axis_index("c")
    s = lax.axis_index("s")
    w = c * NTILE + s
    pltpu.sync_copy(ei_hbm.at[pl.ds(E + w * EPT, EPT)], dvm.at[pl.ds(0, EPT)])

    zv = jnp.zeros((LANES,), jnp.float32)

    def zbody(i, _):
        hist[pl.ds(i * LANES, LANES)] = zv
        return 0

    lax.fori_loop(0, NPAD // LANES, zbody, 0)

    ones = jnp.ones((LANES,), jnp.float32)
    nfull = EPT // LANES

    def abody(i, _):
        idx = dvm[pl.ds(i * LANES, LANES)]
        plsc.addupdate_scatter(hist, [idx], ones)
        return 0

    lax.fori_loop(0, nfull, abody, 0)
    rem = EPT - nfull * LANES
    if rem:
        lanemask = lax.iota(jnp.int32, LANES) < rem
        idx = dvm[pl.ds(nfull * LANES, LANES)]
        idx = jnp.where(lanemask, idx, 0)
        plsc.addupdate_scatter(hist, [idx], ones, mask=lanemask)

    pltpu.sync_copy(hist, shared.at[s])
    plsc.subcore_barrier()
    for r in range(NTILE):
        pltpu.sync_copy(shared.at[r, pl.ds(s * CW, CW)], redbuf.at[r])

    def rbody(k, _):
        acc = redbuf[0, pl.ds(k * LANES, LANES)]
        for r in range(1, NTILE):
            acc = acc + redbuf[r, pl.ds(k * LANES, LANES)]
        sumbuf[pl.ds(k * LANES, LANES)] = acc
        return 0

    lax.fori_loop(0, CW // LANES, rbody, 0)
    pltpu.sync_copy(sumbuf, out_hbm.at[c, pl.ds(s * CW, CW)])


def _make_agg(num_chunks):
    cpc = num_chunks // NSC

    @functools.partial(
        pl.kernel,
        out_type=jax.ShapeDtypeStruct((num_chunks * NPAD, 128), jnp.float32),
        mesh=_mesh(),
        scratch_types=[
            pltpu.VMEM((SG,), jnp.int32),
            pltpu.VMEM((SG,), jnp.int32),
            pltpu.VMEM((SG,), jnp.int32),
            pltpu.VMEM((SG,), jnp.int32),
            pltpu.VMEM((K,), jnp.int32),
            pltpu.VMEM((K,), jnp.int32),
            pltpu.VMEM((K,), jnp.int32),
            pltpu.VMEM((K,), jnp.int32),
            pltpu.VMEM((K,), jnp.int32),
            pltpu.VMEM((K,), jnp.int32),
            pltpu.VMEM((K, 128), jnp.float32),
            pltpu.VMEM((K, 128), jnp.float32),
            pltpu.VMEM((K, 128), jnp.float32),
            pltpu.VMEM_SHARED((NPAD, 128), jnp.float32),
            pltpu.SemaphoreType.DMA,
            pltpu.SemaphoreType.DMA,
            pltpu.SemaphoreType.DMA,
            pltpu.SemaphoreType.DMA,
            pltpu.SemaphoreType.DMA,
        ],
    )
    def agg(y_hbm, ei_hbm, z_hbm, out_hbm, sfa, dfa, sfb, dfb, si0, di0,
            si1, di1, si2, di2, gb0, gb1, gb2, acc, sm0, sm1, sm2, zsem,
            stgsem):
        c = lax.axis_index("c")
        s = lax.axis_index("s")
        sx = [si0, si1, si2]
        dx = [di0, di1, di2]
        gb = [gb0, gb1, gb2]
        sm = [sm0, sm1, sm2]
        sfs = [sfa, sfb]
        dfs = [dfa, dfb]

        for p in range(cpc):
            chunk = c * cpc + p
            base = chunk * NPAD

            def fill(j, t, sf, df):
                e0 = j * K
                for k in range(K // LANES):
                    sv = sf[pl.ds(e0 + k * LANES, LANES)]
                    sx[t][pl.ds(k * LANES, LANES)] = sv + base
                    dx[t][pl.ds(k * LANES, LANES)] = (
                        df[pl.ds(e0 + k * LANES, LANES)])

            def issue(t):
                pltpu.async_copy(y_hbm.at[sx[t]], gb[t], sm[t])

            def drain(t):
                pltpu.make_async_copy(y_hbm.at[sx[t]], gb[t], sm[t]).wait()
                pltpu.sync_copy(gb[t], acc.at[dx[t]], add=True)

            zdesc = pltpu.async_copy(z_hbm.at[pl.ds(s * RPT, RPT)],
                                     acc.at[pl.ds(s * RPT, RPT)], zsem)

            e0p = s * EPS
            pltpu.sync_copy(ei_hbm.at[pl.ds(e0p, SG)], sfa)
            pltpu.sync_copy(ei_hbm.at[pl.ds(E + e0p, SG)], dfa)
            for g in range(NG):
                sf, df = sfs[g % 2], dfs[g % 2]
                sdescs = []
                if g + 1 < NG:
                    g1 = s * EPS + (g + 1) * SG
                    sfn, dfn = sfs[(g + 1) % 2], dfs[(g + 1) % 2]
                    sdescs.append(pltpu.async_copy(
                        ei_hbm.at[pl.ds(g1, SG)], sfn, stgsem))
                    sdescs.append(pltpu.async_copy(
                        ei_hbm.at[pl.ds(E + g1, SG)], dfn, stgsem))
                for t in range(3):
                    fill(t, t, sf, df)
                    issue(t)
                if g == 0:
                    zdesc.wait()
                    plsc.subcore_barrier()

                def ibody(m, _):
                    for t in range(3):
                        drain(t)
                        fill(3 * m + t + 3, t, sf, df)
                        issue(t)
                    return 0

                lax.fori_loop(0, GB // 3 - 1, ibody, 0)
                drain(0)
                fill(GB - 1, 0, sf, df)
                issue(0)
                drain(1)
                drain(2)
                drain(0)
                for sd in sdescs:
                    sd.wait()

            plsc.subcore_barrier()
            pltpu.sync_copy(
                acc.at[pl.ds(s * RPT, RPT)],
                out_hbm.at[pl.ds(base + s * RPT, RPT)])

    return agg


_agg2 = _make_agg(2)
_agg4 = _make_agg(4)


def _prep_body(x_ref, dp_ref, y1_ref, dis_ref):
    d = dp_ref[0, :] + dp_ref[1, :] + 1.0
    dis = 1.0 / jnp.sqrt(d)
    dis_ref[...] = dis
    xb = x_ref[...]
    y1_ref[0] = xb[:, :128] * dis[:, None]
    y1_ref[1] = xb[:, 128:] * dis[:, None]


_prep = pl.pallas_call(
    _prep_body,
    grid=(GRID,),
    in_specs=[
        pl.BlockSpec((RB, D_IN), lambda i: (i, 0)),
        pl.BlockSpec((NSC, RB), lambda i: (0, i)),
    ],
    out_specs=[
        pl.BlockSpec((NSC, RB, 128), lambda i: (0, i, 0)),
        pl.BlockSpec((RB,), lambda i: (i,)),
    ],
    out_shape=[
        jax.ShapeDtypeStruct((NSC, NPAD, 128), jnp.float32),
        jax.ShapeDtypeStruct((N,), jnp.float32),
    ],
)


def _mid_body(a1_ref, y1_ref, dis_ref, w1_ref, b1_ref, w2_ref, y2_ref):
    dis = dis_ref[...][:, None]
    z = jnp.concatenate(
        [(a1_ref[0] + y1_ref[0]) * dis, (a1_ref[1] + y1_ref[1]) * dis],
        axis=1).astype(jnp.bfloat16)
    h = jnp.dot(z, w1_ref[...], preferred_element_type=jnp.float32) + b1_ref[...][None, :]
    h = jnp.maximum(h, 0.0).astype(jnp.bfloat16)
    t = jnp.dot(h, w2_ref[...], preferred_element_type=jnp.float32)
    for q in range(4):
        y2_ref[q] = t[:, q * 128:(q + 1) * 128] * dis


_mid = pl.pallas_call(
    _mid_body,
    grid=(GRID,),
    in_specs=[
        pl.BlockSpec((NSC, RB, 128), lambda i: (0, i, 0)),
        pl.BlockSpec((NSC, RB, 128), lambda i: (0, i, 0)),
        pl.BlockSpec((RB,), lambda i: (i,)),
        pl.BlockSpec((D_IN, HID), lambda i: (0, 0)),
        pl.BlockSpec((HID,), lambda i: (0,)),
        pl.BlockSpec((HID, HID2), lambda i: (0, 0)),
    ],
    out_specs=pl.BlockSpec((4, RB, 128), lambda i: (0, i, 0)),
    out_shape=jax.ShapeDtypeStruct((4, NPAD, 128), jnp.float32),
)


def _fin_body(a2_ref, y2_ref, dis_ref, b2_ref, wc_ref, bc_ref, out_ref):
    dis = dis_ref[...][:, None]
    z = jnp.concatenate(
        [(a2_ref[q] + y2_ref[q]) * dis for q in range(4)], axis=1)
    h = jnp.maximum(z + b2_ref[...][None, :], 0.0)
    logits = jnp.dot(h, wc_ref[...], preferred_element_type=jnp.float32) + bc_ref[...][None, :]
    m = jnp.max(logits, axis=1, keepdims=True)
    ex = jnp.exp(logits - m)
    lse = jnp.log(jnp.sum(ex, axis=1, keepdims=True)) + m
    out_ref[...] = logits - lse


_fin = pl.pallas_call(
    _fin_body,
    grid=(GRID,),
    in_specs=[
        pl.BlockSpec((4, RB, 128), lambda i: (0, i, 0)),
        pl.BlockSpec((4, RB, 128), lambda i: (0, i, 0)),
        pl.BlockSpec((RB,), lambda i: (i,)),
        pl.BlockSpec((HID2,), lambda i: (0,)),
        pl.BlockSpec((HID2, NCLS), lambda i: (0, 0)),
        pl.BlockSpec((NCLS,), lambda i: (0,)),
    ],
    out_specs=pl.BlockSpec((RB, NCLS), lambda i: (i, 0)),
    out_shape=jax.ShapeDtypeStruct((N, NCLS), jnp.float32),
)


def kernel(x, edge_index, W1, b1, W2, b2, Wc, bc):
    ei_flat = edge_index.reshape(2 * E)
    zeros = jnp.zeros((NPAD, 128), jnp.float32)
    deg_p = _deg(ei_flat)
    y1, dis = _prep(x, deg_p)
    agg1 = _agg2(y1.reshape(NSC * NPAD, 128), ei_flat, zeros)
    y2 = _mid(agg1.reshape(NSC, NPAD, 128), y1, dis,
              W1.astype(jnp.bfloat16), b1, W2.astype(jnp.bfloat16))
    agg2 = _agg4(y2.reshape(4 * NPAD, 128), ei_flat, zeros)
    return _fin(agg2.reshape(4, NPAD, 128), y2, dis, b2, Wc, bc)
